# baseline (speedup 1.0000x reference)
import functools

import jax
import jax.numpy as jnp
from jax import lax
from jax.experimental import pallas as pl
from jax.experimental.pallas import tpu as pltpu
from jax.experimental.pallas import tpu_sc as plsc

_NBUF = 8


@functools.lru_cache(maxsize=None)
def _build(n_seq: int, seq_len: int, dim: int):
    info = plsc.get_sparse_core_info()
    nw = info.num_cores * info.num_subcores
    s_per_w = n_seq // nw
    assert s_per_w * nw == n_seq and s_per_w >= _NBUF and seq_len <= 128
    mesh = plsc.VectorSubcoreMesh(core_axis_name="c", subcore_axis_name="s")

    @functools.partial(
        pl.kernel,
        mesh=mesh,
        compiler_params=pltpu.CompilerParams(use_tc_tiling_on_sc=True),
        out_type=jax.ShapeDtypeStruct((n_seq, seq_len, dim), jnp.float32),
        scratch_types=[
            pltpu.VMEM((s_per_w, seq_len), jnp.int32),
            pltpu.VMEM((_NBUF, seq_len, dim), jnp.float32),
            pltpu.SemaphoreType.DMA,
            pltpu.SemaphoreType.DMA,
        ],
    )
    def gather_kernel(idx_hbm, table_hbm, out_hbm, idx_v, rows_v, gsem, ssem):
        wid = lax.axis_index("s") * info.num_cores + lax.axis_index("c")
        base = wid * s_per_w
        pltpu.sync_copy(idx_hbm.at[wid], idx_v)

        for b in range(_NBUF - 1):
            pltpu.async_copy(table_hbm.at[idx_v.at[b]], rows_v.at[b], gsem)

        def body(i, _):
            slot = lax.rem(i, _NBUF)
            pltpu.make_async_copy(
                table_hbm.at[idx_v.at[i]], rows_v.at[slot], gsem
            ).wait()
            pltpu.async_copy(rows_v.at[slot], out_hbm.at[base + i], ssem)
            j = i + _NBUF - 1

            @pl.when(jnp.logical_and(i >= 1, j < s_per_w))
            def _():
                pltpu.make_async_copy(
                    rows_v.at[0], out_hbm.at[base], ssem
                ).wait()

            @pl.when(j < s_per_w)
            def _():
                pltpu.async_copy(
                    table_hbm.at[idx_v.at[j]], rows_v.at[lax.rem(j, _NBUF)], gsem
                )

            return 0

        lax.fori_loop(0, s_per_w, body, 0)
        for _ in range(_NBUF):
            pltpu.make_async_copy(rows_v.at[0], out_hbm.at[base], ssem).wait()

    return gather_kernel


def kernel(token_ids, weight):
    b, s = token_ids.shape
    info = plsc.get_sparse_core_info()
    nw = info.num_cores * info.num_subcores
    idx = jnp.asarray(token_ids, jnp.int32).reshape(nw, b // nw, s)
    return _build(b, s, weight.shape[1])(idx, weight)


# speedup vs baseline: 1.7642x; 1.7642x over previous
import functools

import jax
import jax.numpy as jnp
from jax import lax
from jax.experimental import pallas as pl
from jax.experimental.pallas import tpu as pltpu
from jax.experimental.pallas import tpu_sc as plsc

_IDXW = 128
_NBUF = 6


@functools.lru_cache(maxsize=None)
def _build(num_tokens: int, dim: int):
    info = plsc.get_sparse_core_info()
    nw = info.num_cores * info.num_subcores
    b_per_w = num_tokens // nw
    n_chunks = b_per_w // _IDXW
    assert n_chunks * _IDXW * nw == num_tokens and n_chunks >= _NBUF
    mesh = plsc.VectorSubcoreMesh(core_axis_name="c", subcore_axis_name="s")

    @functools.partial(
        pl.kernel,
        mesh=mesh,
        out_type=jax.ShapeDtypeStruct((num_tokens, dim), jnp.float32),
        scratch_types=[
            pltpu.VMEM((n_chunks, _IDXW), jnp.int32),
            pltpu.VMEM((_NBUF, _IDXW, dim), jnp.float32),
            pltpu.SemaphoreType.DMA,
            pltpu.SemaphoreType.DMA,
        ],
    )
    def gather_kernel(idx_hbm, table_hbm, out_hbm, idx_v, rows_v, gsem, ssem):
        wid = lax.axis_index("s") * info.num_cores + lax.axis_index("c")
        base = wid * b_per_w
        pltpu.sync_copy(idx_hbm.at[wid], idx_v)

        for b in range(_NBUF - 1):
            pltpu.async_copy(table_hbm.at[idx_v.at[b]], rows_v.at[b], gsem)

        def body(i, _):
            slot = lax.rem(i, _NBUF)
            pltpu.make_async_copy(
                table_hbm.at[idx_v.at[i]], rows_v.at[slot], gsem
            ).wait()
            pltpu.async_copy(
                rows_v.at[slot], out_hbm.at[pl.ds(base + i * _IDXW, _IDXW)], ssem
            )
            j = i + _NBUF - 1

            @pl.when(jnp.logical_and(i >= 1, j < n_chunks))
            def _():
                pltpu.make_async_copy(
                    rows_v.at[0], out_hbm.at[pl.ds(base, _IDXW)], ssem
                ).wait()

            @pl.when(j < n_chunks)
            def _():
                pltpu.async_copy(
                    table_hbm.at[idx_v.at[j]], rows_v.at[lax.rem(j, _NBUF)], gsem
                )

            return 0

        lax.fori_loop(0, n_chunks, body, 0)
        for _ in range(_NBUF):
            pltpu.make_async_copy(
                rows_v.at[0], out_hbm.at[pl.ds(base, _IDXW)], ssem
            ).wait()

    return gather_kernel


def kernel(token_ids, weight):
    b, s = token_ids.shape
    num_tokens = b * s
    dim = weight.shape[1]
    info = plsc.get_sparse_core_info()
    nw = info.num_cores * info.num_subcores
    idx = jnp.asarray(token_ids, jnp.int32).T.reshape(
        nw, num_tokens // (nw * _IDXW), _IDXW
    )
    out = _build(num_tokens, dim)(idx, weight)
    return out.reshape(s, b, dim).transpose(1, 0, 2)


# speedup vs baseline: 1.7706x; 1.0036x over previous
import functools

import jax
import jax.numpy as jnp
from jax import lax
from jax.experimental import pallas as pl
from jax.experimental.pallas import tpu as pltpu
from jax.experimental.pallas import tpu_sc as plsc

_IDXW = 128
_NPAIR = 3


@functools.lru_cache(maxsize=None)
def _build(num_tokens: int, dim: int):
    info = plsc.get_sparse_core_info()
    nw = info.num_cores * info.num_subcores
    b_per_w = num_tokens // nw
    n_chunks = b_per_w // _IDXW
    n_pairs = n_chunks // 2
    assert n_pairs * 2 * _IDXW * nw == num_tokens and n_pairs >= _NPAIR + 1
    mesh = plsc.VectorSubcoreMesh(core_axis_name="c", subcore_axis_name="s")

    @functools.partial(
        pl.kernel,
        mesh=mesh,
        out_type=jax.ShapeDtypeStruct((num_tokens, dim), jnp.float32),
        scratch_types=[
            pltpu.VMEM((n_chunks, _IDXW), jnp.int32),
            pltpu.VMEM((_NPAIR, 2 * _IDXW, dim), jnp.float32),
            pltpu.SemaphoreType.DMA,
            pltpu.SemaphoreType.DMA,
        ],
    )
    def gather_kernel(idx_hbm, table_hbm, out_hbm, idx_v, rows_v, gsem, ssem):
        wid = lax.axis_index("s") * info.num_cores + lax.axis_index("c")
        base = wid * b_per_w
        pltpu.sync_copy(idx_hbm.at[wid], idx_v)

        for k in range(4):
            pltpu.async_copy(
                table_hbm.at[idx_v.at[k]],
                rows_v.at[k // 2, pl.ds((k % 2) * _IDXW, _IDXW)],
                gsem,
            )

        def body(p, _):
            pair = lax.rem(p, _NPAIR)
            for k in range(2):
                pltpu.make_async_copy(
                    table_hbm.at[idx_v.at[2 * p + k]],
                    rows_v.at[pair, pl.ds(k * _IDXW, _IDXW)],
                    gsem,
                ).wait()
            pltpu.async_copy(
                rows_v.at[pair],
                out_hbm.at[pl.ds(base + p * 2 * _IDXW, 2 * _IDXW)],
                ssem,
            )
            q = p + _NPAIR - 1

            @pl.when(jnp.logical_and(p >= 1, q < n_pairs))
            def _():
                pltpu.make_async_copy(
                    rows_v.at[0], out_hbm.at[pl.ds(base, 2 * _IDXW)], ssem
                ).wait()

            @pl.when(q < n_pairs)
            def _():
                for k in range(2):
                    pltpu.async_copy(
                        table_hbm.at[idx_v.at[2 * q + k]],
                        rows_v.at[lax.rem(q, _NPAIR), pl.ds(k * _IDXW, _IDXW)],
                        gsem,
                    )

            return 0

        lax.fori_loop(0, n_pairs, body, 0)
        for _ in range(_NPAIR):
            pltpu.make_async_copy(
                rows_v.at[0], out_hbm.at[pl.ds(base, 2 * _IDXW)], ssem
            ).wait()

    return gather_kernel


def kernel(token_ids, weight):
    b, s = token_ids.shape
    num_tokens = b * s
    dim = weight.shape[1]
    info = plsc.get_sparse_core_info()
    nw = info.num_cores * info.num_subcores
    idx = jnp.asarray(token_ids, jnp.int32).T.reshape(
        nw, num_tokens // (nw * _IDXW), _IDXW
    )
    out = _build(num_tokens, dim)(idx, weight)
    return out.reshape(s, b, dim).transpose(1, 0, 2)
